# baseline (speedup 1.0000x reference)
import functools

import jax
import jax.numpy as jnp
from jax import lax
from jax.experimental import pallas as pl
from jax.experimental.pallas import tpu as pltpu
from jax.experimental.pallas import tpu_sc as plsc

NUM_CORES = 2
NUM_SUBCORES = 16
NW = NUM_CORES * NUM_SUBCORES
LANES = 16


def _build(B, S, V, D):
    assert D == 64 and B % (NW * 512) == 0
    bw = B // NW
    C = 256
    nch = S * bw // C
    jw = bw // 128

    mesh = plsc.VectorSubcoreMesh(
        core_axis_name="c", subcore_axis_name="s",
        num_cores=NUM_CORES, num_subcores=NUM_SUBCORES)

    @functools.partial(
        pl.kernel,
        out_type=jax.ShapeDtypeStruct((S, 8, B // 128, 8, 128), jnp.float32),
        mesh=mesh,
        scratch_types=[
            pltpu.VMEM((S, 2, C), jnp.int32),
            pltpu.VMEM((2, C, D), jnp.float32),
            pltpu.VMEM((2, 8, 2, 8, 128), jnp.float32),
            pltpu.SemaphoreType.DMA,
            pltpu.SemaphoreType.DMA,
            pltpu.SemaphoreType.DMA,
            pltpu.SemaphoreType.DMA,
            pltpu.SemaphoreType.DMA,
        ],
        compiler_params=pltpu.CompilerParams(use_tc_tiling_on_sc=False,
                                             needs_layout_passes=False),
    )
    def emb(weight_hbm, idx_hbm, out_hbm, idx_v, gbuf, tbuf,
            isem, gsem0, gsem1, osem0, osem1):
        wid = lax.axis_index("s") * NUM_CORES + lax.axis_index("c")
        gsem = (gsem0, gsem1)
        osem = (osem0, osem1)
        pltpu.async_copy(idx_hbm.at[:, pl.ds(wid * 2, 2), :], idx_v, isem).wait()

        iota = lax.iota(jnp.int32, LANES)

        def gather(m, b):
            s, half = m // 2, m % 2
            pltpu.async_copy(weight_hbm.at[idx_v.at[s, half]],
                             gbuf.at[b], gsem[b])

        def wait_gather(m, b):
            s, half = m // 2, m % 2
            pltpu.make_async_copy(weight_hbm.at[idx_v.at[s, half]],
                                  gbuf.at[b], gsem[b]).wait()

        def out_slice(m):
            s, half = m // 2, m % 2
            return out_hbm.at[s, :, pl.ds(wid * jw + half * 2, 2), :, :]

        def store(m, b):
            pltpu.async_copy(tbuf.at[b], out_slice(m), osem[b])

        def wait_store(m, b):
            pltpu.make_async_copy(tbuf.at[b], out_slice(m), osem[b]).wait()

        rows_kb = [iota + kb * 16 for kb in range(16)]

        def transpose(b):
            @plsc.parallel_loop(0, D, 1, unroll=8)
            def comp(c):
                i, r = c >> 3, c & 7
                cols = jnp.broadcast_to(c, (LANES,))
                for kb in range(16):
                    vals = plsc.load_gather(gbuf.at[b], [rows_kb[kb], cols])
                    jloc, blk = kb // 8, kb % 8
                    tbuf[b, i, jloc, r, pl.ds(blk * 16, 16)] = vals

        gather(0, 0)
        wait_gather(0, 0)
        gather(1, 1)
        transpose(0)
        store(0, 0)
        wait_gather(1, 1)
        gather(2, 0)
        transpose(1)
        store(1, 1)

        def step(gp, _):
            for b in range(2):
                m = 2 + gp * 2 + b
                wait_store(m - 2, b)
                wait_gather(m, b)
                gather(m + 1, 1 - b)
                transpose(b)
                store(m, b)
            return _

        lax.fori_loop(0, (nch - 2) // 2 - 1, step, 0, unroll=False)

        m = nch - 2
        wait_store(m - 2, 0)
        wait_gather(m, 0)
        gather(nch - 1, 1)
        transpose(0)
        store(m, 0)
        m = nch - 1
        wait_store(m - 2, 1)
        wait_gather(m, 1)
        transpose(1)
        store(m, 1)
        wait_store(nch - 2, 0)
        wait_store(nch - 1, 1)

    return emb


def kernel(token_ids, weight):
    Bt, S = token_ids.shape
    V, D = weight.shape
    B = Bt * S
    idxT = jnp.reshape(jnp.transpose(token_ids).astype(jnp.int32), (S, Bt // 256, 256))
    out5 = _build(Bt, S, V, D)(weight, idxT)
    return jnp.reshape(jnp.transpose(out5, (2, 4, 0, 1, 3)), (Bt, S, D))


# speedup vs baseline: 1.4758x; 1.4758x over previous
import functools

import jax
import jax.numpy as jnp
from jax import lax
from jax.experimental import pallas as pl
from jax.experimental.pallas import tpu as pltpu
from jax.experimental.pallas import tpu_sc as plsc

NUM_CORES = 2
NUM_SUBCORES = 16
NW = NUM_CORES * NUM_SUBCORES
LANES = 16


def _build(B, S, V, D):
    assert D == 64 and B % (NW * 512) == 0
    bw = B // NW
    C = 256
    nch = S * bw // C
    jw = bw // 128

    mesh = plsc.VectorSubcoreMesh(
        core_axis_name="c", subcore_axis_name="s",
        num_cores=NUM_CORES, num_subcores=NUM_SUBCORES)

    @functools.partial(
        pl.kernel,
        out_type=jax.ShapeDtypeStruct((S, 8, B // 128, 8, 128), jnp.float32),
        mesh=mesh,
        scratch_types=[
            pltpu.VMEM((S, 2, C), jnp.int32),
            pltpu.VMEM((2, C, D), jnp.float32),
            pltpu.VMEM((2, 8, 2, 8, 128), jnp.float32),
            pltpu.SemaphoreType.DMA,
            pltpu.SemaphoreType.DMA,
            pltpu.SemaphoreType.DMA,
            pltpu.SemaphoreType.DMA,
            pltpu.SemaphoreType.DMA,
        ],
        compiler_params=pltpu.CompilerParams(use_tc_tiling_on_sc=False,
                                             needs_layout_passes=False),
    )
    def emb(weight_hbm, idx_hbm, out_hbm, idx_v, gbuf, tbuf,
            isem, gsem0, gsem1, osem0, osem1):
        wid = lax.axis_index("s") * NUM_CORES + lax.axis_index("c")
        gsem = (gsem0, gsem1)
        osem = (osem0, osem1)
        pltpu.async_copy(idx_hbm.at[:, pl.ds(wid * 2, 2), :], idx_v, isem).wait()

        iota = lax.iota(jnp.int32, LANES)

        def gather(m, b):
            s, half = m // 2, m % 2
            pltpu.async_copy(weight_hbm.at[idx_v.at[s, half]],
                             gbuf.at[b], gsem[b])

        def wait_gather(m, b):
            s, half = m // 2, m % 2
            pltpu.make_async_copy(weight_hbm.at[idx_v.at[s, half]],
                                  gbuf.at[b], gsem[b]).wait()

        def out_slice(m):
            s, half = m // 2, m % 2
            return out_hbm.at[s, :, pl.ds(wid * jw + half * 2, 2), :, :]

        def store(m, b):
            pltpu.async_copy(tbuf.at[b], out_slice(m), osem[b])

        def wait_store(m, b):
            pltpu.make_async_copy(tbuf.at[b], out_slice(m), osem[b]).wait()

        rows_kb = [iota + kb * 16 for kb in range(16)]

        def transpose(b):
            @plsc.parallel_loop(0, LANES, 1, unroll=1)
            def diag(d):
                perm = (iota + d) & 15
                i_p = perm >> 3
                r_vec = perm & 7
                for kb in range(16):
                    jloc, blk = kb // 8, kb % 8
                    jloc_vec = jnp.broadcast_to(jloc, (LANES,))
                    bl_vec = iota + blk * 16
                    for q in range(4):
                        cols = perm + (q * 16)
                        vals = plsc.load_gather(gbuf.at[b],
                                                [rows_kb[kb], cols])
                        i_vec = i_p + (q * 2)
                        plsc.store_scatter(tbuf.at[b],
                                           [i_vec, jloc_vec, r_vec, bl_vec],
                                           vals)

        gather(0, 0)
        wait_gather(0, 0)
        gather(1, 1)
        transpose(0)
        store(0, 0)
        wait_gather(1, 1)
        gather(2, 0)
        transpose(1)
        store(1, 1)

        def step(gp, _):
            for b in range(2):
                m = 2 + gp * 2 + b
                wait_store(m - 2, b)
                wait_gather(m, b)
                gather(m + 1, 1 - b)
                transpose(b)
                store(m, b)
            return _

        lax.fori_loop(0, (nch - 2) // 2 - 1, step, 0, unroll=False)

        m = nch - 2
        wait_store(m - 2, 0)
        wait_gather(m, 0)
        gather(nch - 1, 1)
        transpose(0)
        store(m, 0)
        m = nch - 1
        wait_store(m - 2, 1)
        wait_gather(m, 1)
        transpose(1)
        store(m, 1)
        wait_store(nch - 2, 0)
        wait_store(nch - 1, 1)

    return emb


def kernel(token_ids, weight):
    Bt, S = token_ids.shape
    V, D = weight.shape
    B = Bt * S
    idxT = jnp.reshape(jnp.transpose(token_ids).astype(jnp.int32), (S, Bt // 256, 256))
    out5 = _build(Bt, S, V, D)(weight, idxT)
    return jnp.reshape(jnp.transpose(out5, (2, 4, 0, 1, 3)), (Bt, S, D))
